# Initial kernel scaffold; baseline (speedup 1.0000x reference)
import jax
import jax.numpy as jnp
from jax.experimental import pallas as pl


def kernel(x, w1, b1, w2, b2, w_fc1, b_fc1, w_fc2, b_fc2, w_fc3, b_fc3):
    raise NotImplementedError("write your pallas kernel here")


# speedup vs baseline: 2.4051x; 2.4051x over previous
import functools

import jax
import jax.numpy as jnp
from jax.experimental import pallas as pl
from jax.experimental.pallas import tpu as pltpu

CDT = jnp.bfloat16


def _conv1_pool_kernel(p00_ref, p01_ref, p10_ref, p11_ref, w_ref, b_ref,
                       o_ref):
    w = w_ref[...]
    f32 = jnp.float32
    y = jnp.maximum(
        jnp.maximum(jnp.dot(p00_ref[...], w, preferred_element_type=f32),
                    jnp.dot(p01_ref[...], w, preferred_element_type=f32)),
        jnp.maximum(jnp.dot(p10_ref[...], w, preferred_element_type=f32),
                    jnp.dot(p11_ref[...], w, preferred_element_type=f32)))
    y = jnp.maximum(y + b_ref[...], 0.0).astype(o_ref.dtype)
    bt, rows, c = o_ref.shape
    o_ref[:, :841, :] = y.reshape(bt, 841, c)
    o_ref[:, 841:, :] = jnp.zeros((bt, rows - 841, c), o_ref.dtype)


def _conv1_pool(parities, w1, b1, *, n, bt):
    grid = (n // bt,)
    m = bt * 841
    return pl.pallas_call(
        _conv1_pool_kernel,
        out_shape=jax.ShapeDtypeStruct((n, 870, 64), CDT),
        grid_spec=pltpu.PrefetchScalarGridSpec(
            num_scalar_prefetch=0,
            grid=grid,
            in_specs=[pl.BlockSpec((m, 25), lambda i: (i, 0))] * 4 +
                     [pl.BlockSpec((25, 64), lambda i: (0, 0)),
                      pl.BlockSpec((1, 64), lambda i: (0, 0))],
            out_specs=pl.BlockSpec((bt, 870, 64), lambda i: (i, 0, 0))),
        compiler_params=pltpu.CompilerParams(
            dimension_semantics=("parallel",)),
    )(*parities, w1, b1)


def _conv2_kernel(x_ref, w_ref, b_ref, o_ref, *, bt, wide):
    f32 = jnp.float32
    for img in range(bt):
        acc = None
        for t in range(25):
            i, j = divmod(t, 5)
            off = i * 29 + j
            part = jnp.dot(x_ref[img, off:off + wide, :], w_ref[t],
                           preferred_element_type=f32)
            acc = part if acc is None else acc + part
        o_ref[img, :, :] = jnp.maximum(acc + b_ref[...], 0.0).astype(o_ref.dtype)


def _conv2(pooled, w2, b2, *, n, bt, wide):
    kfn = functools.partial(_conv2_kernel, bt=bt, wide=wide)
    return pl.pallas_call(
        kfn,
        out_shape=jax.ShapeDtypeStruct((n, wide, 64), CDT),
        grid_spec=pltpu.PrefetchScalarGridSpec(
            num_scalar_prefetch=0,
            grid=(n // bt,),
            in_specs=[pl.BlockSpec((bt, 870, 64), lambda i: (i, 0, 0)),
                      pl.BlockSpec((25, 64, 64), lambda i: (0, 0, 0)),
                      pl.BlockSpec((1, 64), lambda i: (0, 0))],
            out_specs=pl.BlockSpec((bt, wide, 64), lambda i: (i, 0, 0))),
        compiler_params=pltpu.CompilerParams(
            dimension_semantics=("parallel",)),
    )(pooled, w2, b2)


def _fc_kernel(x_ref, w1_ref, b1_ref, w2_ref, b2_ref, w3_ref, b3_ref,
               o_ref, acc_ref, *, n_k):
    k = pl.program_id(1)

    @pl.when(k == 0)
    def _():
        acc_ref[...] = jnp.zeros_like(acc_ref)

    acc_ref[...] += jnp.dot(x_ref[...], w1_ref[...],
                            preferred_element_type=jnp.float32)

    @pl.when(k == n_k - 1)
    def _():
        h1 = jnp.maximum(acc_ref[...] + b1_ref[...], 0.0)
        h2 = jnp.maximum(jnp.dot(h1, w2_ref[...],
                                 preferred_element_type=jnp.float32)
                         + b2_ref[...], 0.0)
        o_ref[...] = (jnp.dot(h2, w3_ref[...],
                              preferred_element_type=jnp.float32)
                      + b3_ref[...]).astype(o_ref.dtype)


def _fc_tail(xk, w1p, b1, w2, b2, w3, b3, *, n, n_k, tk):
    kfn = functools.partial(_fc_kernel, n_k=n_k)
    bm = n // 2
    return pl.pallas_call(
        kfn,
        out_shape=jax.ShapeDtypeStruct((n, 2), jnp.float32),
        grid_spec=pltpu.PrefetchScalarGridSpec(
            num_scalar_prefetch=0,
            grid=(2, n_k),
            in_specs=[pl.BlockSpec((bm, tk), lambda m, k: (m, k)),
                      pl.BlockSpec((tk, 64), lambda m, k: (k, 0)),
                      pl.BlockSpec((1, 64), lambda m, k: (0, 0)),
                      pl.BlockSpec((64, 128), lambda m, k: (0, 0)),
                      pl.BlockSpec((1, 128), lambda m, k: (0, 0)),
                      pl.BlockSpec((128, 2), lambda m, k: (0, 0)),
                      pl.BlockSpec((1, 2), lambda m, k: (0, 0))],
            out_specs=pl.BlockSpec((bm, 2), lambda m, k: (m, 0)),
            scratch_shapes=[pltpu.VMEM((bm, 64), jnp.float32)]),
        compiler_params=pltpu.CompilerParams(
            dimension_semantics=("parallel", "arbitrary")),
    )(xk, w1p, b1, w2, b2, w3, b3)


def kernel(x, w1, b1, w2, b2, w_fc1, b_fc1, w_fc2, b_fc2, w_fc3, b_fc3):
    n = x.shape[0]
    x = x.reshape(n, 62, 62)
    wide = 728

    parities = []
    for di in (0, 1):
        for dj in (0, 1):
            taps = [x[:, di + i:di + i + 58:2, dj + j:dj + j + 58:2]
                    for i in range(5) for j in range(5)]
            p = jnp.stack(taps, axis=-1).astype(CDT).reshape(n * 841, 25)
            parities.append(p)

    pooled = _conv1_pool(parities, w1.astype(CDT), b1, n=n, bt=8)

    yw = _conv2(pooled, w2.astype(CDT), b2, n=n, bt=8, wide=wide)

    wf = w_fc1.reshape(25, 25, 64, 64)
    wf = jnp.pad(wf, ((0, 0), (0, 4), (0, 0), (0, 0)))
    wf = wf.reshape(25 * 29 * 64, 64)
    wf = jnp.pad(wf, ((0, (wide - 725) * 64), (0, 0))).astype(CDT)

    xk = yw.reshape(n, wide * 64)
    n_k = 4
    return _fc_tail(xk, wf, b_fc1, w_fc2, b_fc2, w_fc3, b_fc3,
                    n=n, n_k=n_k, tk=wide * 64 // n_k)


# speedup vs baseline: 6.6999x; 2.7857x over previous
import functools

import jax
import jax.numpy as jnp
from jax.experimental import pallas as pl
from jax.experimental.pallas import tpu as pltpu

CDT = jnp.bfloat16
WPAD = 32
WIDE = 25 * WPAD
ROWS = 30 * WPAD


def _c1_kernel(x_ref, t5_ref, b_ref, o_ref):
    bt = x_ref.shape[0]
    xb = x_ref[...].astype(CDT)
    parts = [xb[:, i:i + 58, :].reshape(bt * 58, 62) for i in range(5)]
    parts.append(jnp.zeros((bt * 58, 10), CDT))
    lhs = jnp.concatenate(parts, axis=-1)
    y = jnp.dot(lhs, t5_ref[...],
                preferred_element_type=jnp.float32)
    y = y.reshape(bt * 58, 29, 128)
    y = jnp.maximum(y[:, :, :64], y[:, :, 64:])
    y = y.reshape(bt * 29, 2, 29, 64)
    y = jnp.maximum(y[:, 0], y[:, 1])
    y = jnp.maximum(y + b_ref[...], 0.0).astype(o_ref.dtype)
    y = y.reshape(bt, 29, 29, 64)
    y = jnp.pad(y, ((0, 0), (0, 1), (0, WPAD - 29), (0, 0)))
    o_ref[...] = y.reshape(bt, ROWS, 64)


def _conv1_pool(x, t5, b1, *, n, bt):
    return pl.pallas_call(
        _c1_kernel,
        out_shape=jax.ShapeDtypeStruct((n, ROWS, 64), CDT),
        grid_spec=pltpu.PrefetchScalarGridSpec(
            num_scalar_prefetch=0,
            grid=(n // bt,),
            in_specs=[pl.BlockSpec((bt, 62, 62), lambda i: (i, 0, 0)),
                      pl.BlockSpec((320, 3712), lambda i: (0, 0)),
                      pl.BlockSpec((1, 64), lambda i: (0, 0))],
            out_specs=pl.BlockSpec((bt, ROWS, 64), lambda i: (i, 0, 0))),
        compiler_params=pltpu.CompilerParams(
            dimension_semantics=("parallel",)),
    )(x, t5, b1)


def _c2_kernel(x_ref, w_ref, b_ref, o_ref, *, bt):
    for img in range(bt):
        xi = x_ref[img]
        x5 = jnp.concatenate([xi[s:s + 936, :] for s in range(5)],
                             axis=-1)
        acc = None
        for i in range(5):
            part = jnp.dot(x5[i * WPAD:i * WPAD + WIDE, :], w_ref[i],
                           preferred_element_type=jnp.float32)
            acc = part if acc is None else acc + part
        o_ref[img] = jnp.maximum(acc + b_ref[...], 0.0).astype(o_ref.dtype)


def _conv2(pooled, w5, b2, *, n, bt):
    kfn = functools.partial(_c2_kernel, bt=bt)
    return pl.pallas_call(
        kfn,
        out_shape=jax.ShapeDtypeStruct((n, WIDE, 64), CDT),
        grid_spec=pltpu.PrefetchScalarGridSpec(
            num_scalar_prefetch=0,
            grid=(n // bt,),
            in_specs=[pl.BlockSpec((bt, ROWS, 64), lambda i: (i, 0, 0)),
                      pl.BlockSpec((5, 320, 64), lambda i: (0, 0, 0)),
                      pl.BlockSpec((1, 64), lambda i: (0, 0))],
            out_specs=pl.BlockSpec((bt, WIDE, 64), lambda i: (i, 0, 0))),
        compiler_params=pltpu.CompilerParams(
            dimension_semantics=("parallel",)),
    )(pooled, w5, b2)


def _fc_kernel(x_ref, w1_ref, b1_ref, w2_ref, b2_ref, w3_ref, b3_ref,
               o_ref, acc_ref, *, n_k):
    k = pl.program_id(1)

    @pl.when(k == 0)
    def _():
        acc_ref[...] = jnp.zeros_like(acc_ref)

    acc_ref[...] += jnp.dot(x_ref[...], w1_ref[...],
                            preferred_element_type=jnp.float32)

    @pl.when(k == n_k - 1)
    def _():
        h1 = jnp.maximum(acc_ref[...] + b1_ref[...], 0.0)
        h2 = jnp.maximum(jnp.dot(h1, w2_ref[...],
                                 preferred_element_type=jnp.float32)
                         + b2_ref[...], 0.0)
        o_ref[...] = (jnp.dot(h2, w3_ref[...],
                              preferred_element_type=jnp.float32)
                      + b3_ref[...]).astype(o_ref.dtype)


def _fc_tail(xk, w1p, b1, w2, b2, w3, b3, *, n, n_k, tk):
    kfn = functools.partial(_fc_kernel, n_k=n_k)
    bm = n // 2
    return pl.pallas_call(
        kfn,
        out_shape=jax.ShapeDtypeStruct((n, 2), jnp.float32),
        grid_spec=pltpu.PrefetchScalarGridSpec(
            num_scalar_prefetch=0,
            grid=(2, n_k),
            in_specs=[pl.BlockSpec((bm, tk), lambda m, k: (m, k)),
                      pl.BlockSpec((tk, 64), lambda m, k: (k, 0)),
                      pl.BlockSpec((1, 64), lambda m, k: (0, 0)),
                      pl.BlockSpec((64, 128), lambda m, k: (0, 0)),
                      pl.BlockSpec((1, 128), lambda m, k: (0, 0)),
                      pl.BlockSpec((128, 2), lambda m, k: (0, 0)),
                      pl.BlockSpec((1, 2), lambda m, k: (0, 0))],
            out_specs=pl.BlockSpec((bm, 2), lambda m, k: (m, 0)),
            scratch_shapes=[pltpu.VMEM((bm, 64), jnp.float32)]),
        compiler_params=pltpu.CompilerParams(
            dimension_semantics=("parallel", "arbitrary")),
    )(xk, w1p, b1, w2, b2, w3, b3)


def kernel(x, w1, b1, w2, b2, w_fc1, b_fc1, w_fc2, b_fc2, w_fc3, b_fc3):
    n = x.shape[0]
    x = x.reshape(n, 62, 62)

    diags = [jnp.eye(62, 58, k=-j, dtype=jnp.float32) for j in range(5)]
    t5 = jnp.stack(
        [sum(diags[j][:, :, None] * w1[i * 5 + j][None, None, :]
             for j in range(5)) for i in range(5)])
    t5 = jnp.pad(t5.reshape(310, 3712), ((0, 10), (0, 0))).astype(CDT)

    pooled = _conv1_pool(x, t5, b1, n=n, bt=8)

    w5 = w2.reshape(5, 320, 64).astype(CDT)
    yw = _conv2(pooled, w5, b2, n=n, bt=8)

    wf = w_fc1.reshape(25, 25, 64, 64)
    wf = jnp.pad(wf, ((0, 0), (0, WPAD - 25), (0, 0), (0, 0)))
    wf = wf.reshape(WIDE * 64, 64).astype(CDT)

    xk = yw.reshape(n, WIDE * 64)
    n_k = 4
    return _fc_tail(xk, wf, b_fc1, w_fc2, b_fc2, w_fc3, b_fc3,
                    n=n, n_k=n_k, tk=WIDE * 64 // n_k)


# speedup vs baseline: 7.3008x; 1.0897x over previous
import functools

import jax
import jax.numpy as jnp
from jax.experimental import pallas as pl
from jax.experimental.pallas import tpu as pltpu

CDT = jnp.bfloat16
WPAD = 32
WIDE = 25 * WPAD
ROWS = 30 * WPAD


def _conv_kernel(x_ref, t5_ref, b1_ref, w5_ref, b2_ref, o_ref, *, bt):
    xb = x_ref[...].astype(CDT)
    parts = [xb[:, i:i + 58, :].reshape(bt * 58, 62) for i in range(5)]
    parts.append(jnp.zeros((bt * 58, 10), CDT))
    lhs = jnp.concatenate(parts, axis=-1)
    y = jnp.dot(lhs, t5_ref[...],
                preferred_element_type=jnp.float32)
    y = y.reshape(bt * 58, 29, 128)
    y = jnp.maximum(y[:, :, :64], y[:, :, 64:])
    y = y.reshape(bt * 29, 2, 29, 64)
    y = jnp.maximum(y[:, 0], y[:, 1])
    y = jnp.maximum(y + b1_ref[...], 0.0).astype(CDT)
    y = y.reshape(bt, 29, 29, 64)
    y = jnp.pad(y, ((0, 0), (0, 1), (0, WPAD - 29), (0, 0)))
    pooled = y.reshape(bt, ROWS, 64)

    for img in range(bt):
        xi = pooled[img]
        x5 = jnp.concatenate([xi[s:s + 936, :] for s in range(5)],
                             axis=-1)
        acc = None
        for i in range(5):
            part = jnp.dot(x5[i * WPAD:i * WPAD + WIDE, :], w5_ref[i],
                           preferred_element_type=jnp.float32)
            acc = part if acc is None else acc + part
        o_ref[img] = jnp.maximum(acc + b2_ref[...], 0.0).astype(o_ref.dtype)


def _conv_stack(x, t5, b1, w5, b2, *, n, bt):
    kfn = functools.partial(_conv_kernel, bt=bt)
    return pl.pallas_call(
        kfn,
        out_shape=jax.ShapeDtypeStruct((n, WIDE, 64), CDT),
        grid_spec=pltpu.PrefetchScalarGridSpec(
            num_scalar_prefetch=0,
            grid=(n // bt,),
            in_specs=[pl.BlockSpec((bt, 62, 62), lambda i: (i, 0, 0)),
                      pl.BlockSpec((320, 3712), lambda i: (0, 0)),
                      pl.BlockSpec((1, 64), lambda i: (0, 0)),
                      pl.BlockSpec((5, 320, 64), lambda i: (0, 0, 0)),
                      pl.BlockSpec((1, 64), lambda i: (0, 0))],
            out_specs=pl.BlockSpec((bt, WIDE, 64), lambda i: (i, 0, 0))),
        compiler_params=pltpu.CompilerParams(
            dimension_semantics=("parallel",)),
    )(x, t5, b1, w5, b2)


def _fc_kernel(x_ref, w1_ref, b1_ref, w2_ref, b2_ref, w3_ref, b3_ref,
               o_ref, acc_ref, *, n_k):
    k = pl.program_id(1)

    @pl.when(k == 0)
    def _():
        acc_ref[...] = jnp.zeros_like(acc_ref)

    acc_ref[...] += jnp.dot(x_ref[...], w1_ref[...],
                            preferred_element_type=jnp.float32)

    @pl.when(k == n_k - 1)
    def _():
        h1 = jnp.maximum(acc_ref[...] + b1_ref[...], 0.0)
        h2 = jnp.maximum(jnp.dot(h1, w2_ref[...],
                                 preferred_element_type=jnp.float32)
                         + b2_ref[...], 0.0)
        o_ref[...] = (jnp.dot(h2, w3_ref[...],
                              preferred_element_type=jnp.float32)
                      + b3_ref[...]).astype(o_ref.dtype)


def _fc_tail(xk, w1p, b1, w2, b2, w3, b3, *, n, n_k, tk):
    kfn = functools.partial(_fc_kernel, n_k=n_k)
    bm = n // 2
    return pl.pallas_call(
        kfn,
        out_shape=jax.ShapeDtypeStruct((n, 2), jnp.float32),
        grid_spec=pltpu.PrefetchScalarGridSpec(
            num_scalar_prefetch=0,
            grid=(2, n_k),
            in_specs=[pl.BlockSpec((bm, tk), lambda m, k: (m, k)),
                      pl.BlockSpec((tk, 64), lambda m, k: (k, 0)),
                      pl.BlockSpec((1, 64), lambda m, k: (0, 0)),
                      pl.BlockSpec((64, 128), lambda m, k: (0, 0)),
                      pl.BlockSpec((1, 128), lambda m, k: (0, 0)),
                      pl.BlockSpec((128, 2), lambda m, k: (0, 0)),
                      pl.BlockSpec((1, 2), lambda m, k: (0, 0))],
            out_specs=pl.BlockSpec((bm, 2), lambda m, k: (m, 0)),
            scratch_shapes=[pltpu.VMEM((bm, 64), jnp.float32)]),
        compiler_params=pltpu.CompilerParams(
            dimension_semantics=("parallel", "arbitrary")),
    )(xk, w1p, b1, w2, b2, w3, b3)


def kernel(x, w1, b1, w2, b2, w_fc1, b_fc1, w_fc2, b_fc2, w_fc3, b_fc3):
    n = x.shape[0]
    x = x.reshape(n, 62, 62)

    diags = [jnp.eye(62, 58, k=-j, dtype=jnp.float32) for j in range(5)]
    t5 = jnp.stack(
        [sum(diags[j][:, :, None] * w1[i * 5 + j][None, None, :]
             for j in range(5)) for i in range(5)])
    t5 = jnp.pad(t5.reshape(310, 3712), ((0, 10), (0, 0))).astype(CDT)

    w5 = w2.reshape(5, 320, 64).astype(CDT)
    yw = _conv_stack(x, t5, b1, w5, b2, n=n, bt=16)

    wf = w_fc1.reshape(25, 25, 64, 64)
    wf = jnp.pad(wf, ((0, 0), (0, WPAD - 25), (0, 0), (0, 0)))
    wf = wf.reshape(WIDE * 64, 64).astype(CDT)

    xk = yw.reshape(n, WIDE * 64)
    n_k = 4
    return _fc_tail(xk, wf, b_fc1, w_fc2, b_fc2, w_fc3, b_fc3,
                    n=n, n_k=n_k, tk=WIDE * 64 // n_k)


# speedup vs baseline: 8.9941x; 1.2319x over previous
import functools

import jax
import jax.numpy as jnp
from jax.experimental import pallas as pl
from jax.experimental.pallas import tpu as pltpu

CDT = jnp.bfloat16
WPAD = 32
WIDE = 25 * WPAD
ROWS = 30 * WPAD


def _conv_kernel(x_ref, t5_ref, b1_ref, w5_ref, b2_ref, o_ref, *, bt):
    xb = x_ref[...].astype(CDT)
    parts = [xb[:, i:i + 58, :].reshape(bt * 58, 62) for i in range(5)]
    parts.append(jnp.zeros((bt * 58, 10), CDT))
    lhs = jnp.concatenate(parts, axis=-1)
    y = jnp.dot(lhs, t5_ref[...],
                preferred_element_type=jnp.float32)
    y = y.reshape(bt * 58, 29, 128)
    y = jnp.maximum(y[:, :, :64], y[:, :, 64:])
    y = y.reshape(bt * 29, 2, 29, 64)
    y = jnp.maximum(y[:, 0], y[:, 1])
    y = jnp.maximum(y + b1_ref[...], 0.0).astype(CDT)
    y = y.reshape(bt, 29, 29, 64)
    y = jnp.pad(y, ((0, 0), (0, 1), (0, WPAD - 29), (0, 0)))
    pooled = y.reshape(bt, ROWS, 64)

    for img in range(bt):
        xi = pooled[img]
        x5 = jnp.concatenate([xi[s:s + 936, :] for s in range(5)],
                             axis=-1)
        acc = None
        for i in range(5):
            part = jnp.dot(x5[i * WPAD:i * WPAD + WIDE, :], w5_ref[i],
                           preferred_element_type=jnp.float32)
            acc = part if acc is None else acc + part
        o_ref[img] = jnp.maximum(acc + b2_ref[...], 0.0).astype(o_ref.dtype)


def _conv_stack(x, t5, b1, w5, b2, *, n, bt):
    kfn = functools.partial(_conv_kernel, bt=bt)
    return pl.pallas_call(
        kfn,
        out_shape=jax.ShapeDtypeStruct((n, WIDE, 64), CDT),
        grid_spec=pltpu.PrefetchScalarGridSpec(
            num_scalar_prefetch=0,
            grid=(n // bt,),
            in_specs=[pl.BlockSpec((bt, 62, 62), lambda i: (i, 0, 0)),
                      pl.BlockSpec((320, 3712), lambda i: (0, 0)),
                      pl.BlockSpec((1, 64), lambda i: (0, 0)),
                      pl.BlockSpec((5, 320, 64), lambda i: (0, 0, 0)),
                      pl.BlockSpec((1, 64), lambda i: (0, 0))],
            out_specs=pl.BlockSpec((bt, WIDE, 64), lambda i: (i, 0, 0))),
        compiler_params=pltpu.CompilerParams(
            dimension_semantics=("parallel",)),
    )(x, t5, b1, w5, b2)


def _fc_kernel(x_ref, w1_ref, b1_ref, w2_ref, b2_ref, w3_ref, b3_ref,
               o_ref, acc_ref, *, n_k):
    k = pl.program_id(1)

    @pl.when(k == 0)
    def _():
        acc_ref[...] = jnp.zeros_like(acc_ref)

    bm, rc, c = x_ref.shape
    xc = x_ref[...].reshape(bm, rc * c)
    acc_ref[...] += jnp.dot(xc, w1_ref[...],
                            preferred_element_type=jnp.float32)

    @pl.when(k == n_k - 1)
    def _():
        h1 = jnp.maximum(acc_ref[...] + b1_ref[...], 0.0)
        h2 = jnp.maximum(jnp.dot(h1, w2_ref[...],
                                 preferred_element_type=jnp.float32)
                         + b2_ref[...], 0.0)
        o_ref[...] = (jnp.dot(h2, w3_ref[...],
                              preferred_element_type=jnp.float32)
                      + b3_ref[...]).astype(o_ref.dtype)


def _fc_tail(yw, w1p, b1, w2, b2, w3, b3, *, n, n_k):
    kfn = functools.partial(_fc_kernel, n_k=n_k)
    bm = n // 2
    rc = WIDE // n_k
    tk = rc * 64
    return pl.pallas_call(
        kfn,
        out_shape=jax.ShapeDtypeStruct((n, 2), jnp.float32),
        grid_spec=pltpu.PrefetchScalarGridSpec(
            num_scalar_prefetch=0,
            grid=(2, n_k),
            in_specs=[pl.BlockSpec((bm, rc, 64), lambda m, k: (m, k, 0)),
                      pl.BlockSpec((tk, 64), lambda m, k: (k, 0)),
                      pl.BlockSpec((1, 64), lambda m, k: (0, 0)),
                      pl.BlockSpec((64, 128), lambda m, k: (0, 0)),
                      pl.BlockSpec((1, 128), lambda m, k: (0, 0)),
                      pl.BlockSpec((128, 2), lambda m, k: (0, 0)),
                      pl.BlockSpec((1, 2), lambda m, k: (0, 0))],
            out_specs=pl.BlockSpec((bm, 2), lambda m, k: (m, 0)),
            scratch_shapes=[pltpu.VMEM((bm, 64), jnp.float32)]),
        compiler_params=pltpu.CompilerParams(
            dimension_semantics=("parallel", "arbitrary")),
    )(yw, w1p, b1, w2, b2, w3, b3)


def kernel(x, w1, b1, w2, b2, w_fc1, b_fc1, w_fc2, b_fc2, w_fc3, b_fc3):
    n = x.shape[0]
    x = x.reshape(n, 62, 62)

    diags = jnp.stack([jnp.eye(62, 58, k=-j, dtype=jnp.float32)
                       for j in range(5)])
    t5 = jnp.einsum('jqo,ijc->iqoc', diags, w1.reshape(5, 5, 64))
    t5 = jnp.pad(t5.reshape(310, 3712), ((0, 10), (0, 0))).astype(CDT)

    w5 = w2.reshape(5, 320, 64).astype(CDT)
    yw = _conv_stack(x, t5, b1, w5, b2, n=n, bt=16)

    wf = w_fc1.reshape(25, 25, 64, 64)
    wf = jnp.pad(wf, ((0, 0), (0, WPAD - 25), (0, 0), (0, 0)))
    wf = wf.reshape(WIDE * 64, 64).astype(CDT)

    return _fc_tail(yw, wf, b_fc1, w_fc2, b_fc2, w_fc3, b_fc3, n=n, n_k=4)
